# baseline (speedup 1.0000x reference)
import functools

import numpy as np
import jax
import jax.numpy as jnp
from jax import lax
from jax.experimental import pallas as pl
from jax.experimental.pallas import tpu as pltpu
from jax.experimental.pallas import tpu_sc as plsc

_N = 131072
_V = 4096
_R = 16
_NSUB = 16
_SEG = _N // _NSUB
_GRP = _SEG // 16
_PBLK = 1024
_PICK_SCALE = np.float32(_V - 1) / np.float32(_N - 1)


def _sc_zero(ref, n):
    def body(i, carry):
        ref[pl.ds(i * 16, 16)] = jnp.zeros((16,), ref.dtype)
        return carry
    lax.fori_loop(0, n // 16, body, None)


def _select_body(xyzf_hbm, pick_hbm, out_hbm,
                 xyzf_v, keys_v, locc_v, hist_v, base_v, tot_v, tmp_v, pick_v,
                 mm_v, mm_all_v, vx_v, vy_v, vz_v, acc_v, buf_v,
                 hist_sh, vert_sh, mm_sh):
    t = lax.axis_index("s")
    seg0 = t * _SEG

    pltpu.sync_copy(xyzf_hbm.at[pl.ds(seg0 * 3, _SEG * 3)], xyzf_v)
    pltpu.sync_copy(pick_hbm, pick_v)

    _sc_zero(hist_v, _V)
    _sc_zero(base_v, _V)
    _sc_zero(tot_v, _V)
    _sc_zero(vx_v, _V)
    _sc_zero(vy_v, _V)
    _sc_zero(vz_v, _V)

    lanes = lax.iota(jnp.int32, 16)
    i3 = lanes * 3

    big = jnp.full((16,), 3.4e38, jnp.float32)

    def p0(g, carry):
        ix = i3 + g * 48
        xg = plsc.load_gather(xyzf_v, [ix])
        yg = plsc.load_gather(xyzf_v, [ix + 1])
        zg = plsc.load_gather(xyzf_v, [ix + 2])
        return (jnp.minimum(carry[0], xg), jnp.minimum(carry[1], yg),
                jnp.minimum(carry[2], zg), jnp.minimum(carry[3], -xg),
                jnp.minimum(carry[4], -yg), jnp.minimum(carry[5], -zg))
    accs = lax.fori_loop(0, _GRP, p0, (big, big, big, big, big, big))
    packed = jnp.full((16,), 3.4e38, jnp.float32)
    for k in range(6):
        packed = jnp.where(lanes == k, jnp.min(accs[k]), packed)
    mm_v[...] = packed
    pltpu.sync_copy(mm_v, mm_sh.at[pl.ds(t * 16, 16)])
    plsc.subcore_barrier()
    pltpu.sync_copy(mm_sh, mm_all_v)
    comb = mm_all_v[pl.ds(0, 16)]
    for t2 in range(1, _NSUB):
        comb = jnp.minimum(comb, mm_all_v[pl.ds(t2 * 16, 16)])
    mm_v[...] = comb
    zz = jnp.zeros((16,), jnp.int32)
    mn_x = plsc.load_gather(mm_v, [zz])
    mn_y = plsc.load_gather(mm_v, [zz + 1])
    mn_z = plsc.load_gather(mm_v, [zz + 2])
    cell_x = (-plsc.load_gather(mm_v, [zz + 3]) - mn_x) / _R + 1e-12
    cell_y = (-plsc.load_gather(mm_v, [zz + 4]) - mn_y) / _R + 1e-12
    cell_z = (-plsc.load_gather(mm_v, [zz + 5]) - mn_z) / _R + 1e-12

    def p1(g, carry):
        sl = pl.ds(g * 16, 16)
        ix = i3 + g * 48
        xg = plsc.load_gather(xyzf_v, [ix])
        yg = plsc.load_gather(xyzf_v, [ix + 1])
        zg = plsc.load_gather(xyzf_v, [ix + 2])
        kx = jnp.clip(((xg - mn_x) / cell_x).astype(jnp.int32), 0, _R - 1)
        ky = jnp.clip(((yg - mn_y) / cell_y).astype(jnp.int32), 0, _R - 1)
        kz = jnp.clip(((zg - mn_z) / cell_z).astype(jnp.int32), 0, _R - 1)
        k16 = kx * (_R * _R) + ky * _R + kz
        keys_v[sl] = k16
        gath = plsc.load_gather(hist_v, [k16])
        cnt, last = plsc.scan_count(k16)
        locc_v[sl] = gath + cnt - 1
        plsc.store_scatter(hist_v, [k16], gath + cnt, mask=last)
        return carry
    lax.fori_loop(0, _GRP, p1, None)

    pltpu.sync_copy(hist_v, hist_sh.at[pl.ds(t * _V, _V)])
    plsc.subcore_barrier()
    for t2 in range(_NSUB):
        pltpu.sync_copy(hist_sh.at[pl.ds(t2 * _V, _V)], tmp_v)

        def p2(i, carry, _t2=t2):
            sl = pl.ds(i * 16, 16)
            h = tmp_v[sl]
            tot_v[sl] = tot_v[sl] + h
            base_v[sl] = base_v[sl] + jnp.where(_t2 < t, h, 0)
            return carry
        lax.fori_loop(0, _V // 16, p2, None)

    def p2c(i, carry):
        sl = pl.ds(i * 16, 16)
        v = tot_v[sl]
        cs = plsc.cumsum(v)
        base_v[sl] = base_v[sl] + cs - v + carry
        return carry + jnp.sum(v)
    lax.fori_loop(0, _V // 16, p2c, jnp.int32(0))

    def p3(g, carry):
        sl = pl.ds(g * 16, 16)
        k16 = keys_v[sl]
        rank = plsc.load_gather(base_v, [k16]) + locc_v[sl]
        i0 = (rank.astype(jnp.float32) * _PICK_SCALE).astype(jnp.int32)
        hit_any = jnp.zeros((16,), jnp.bool_)
        isel = jnp.zeros((16,), jnp.int32)
        for dd in (-1, 0, 1):
            ic = jnp.clip(i0 + dd, 0, _V - 1)
            pk = plsc.load_gather(pick_v, [ic])
            hit = jnp.logical_and(pk == rank, jnp.logical_not(hit_any))
            isel = jnp.where(hit, ic, isel)
            hit_any = jnp.logical_or(hit_any, hit)
        ix = i3 + g * 48
        xg = plsc.load_gather(xyzf_v, [ix])
        yg = plsc.load_gather(xyzf_v, [ix + 1])
        zg = plsc.load_gather(xyzf_v, [ix + 2])
        plsc.store_scatter(vx_v, [isel], xg, mask=hit_any)
        plsc.store_scatter(vy_v, [isel], yg, mask=hit_any)
        plsc.store_scatter(vz_v, [isel], zg, mask=hit_any)
        return carry
    lax.fori_loop(0, _GRP, p3, None)

    pltpu.sync_copy(vx_v, vert_sh.at[pl.ds((t * 3 + 0) * _V, _V)])
    pltpu.sync_copy(vy_v, vert_sh.at[pl.ds((t * 3 + 1) * _V, _V)])
    pltpu.sync_copy(vz_v, vert_sh.at[pl.ds((t * 3 + 2) * _V, _V)])
    plsc.subcore_barrier()
    vslc = _V // _NSUB
    for c in range(3):
        _sc_zero(acc_v, vslc)
        for t2 in range(_NSUB):
            pltpu.sync_copy(
                vert_sh.at[pl.ds((t2 * 3 + c) * _V + t * vslc, vslc)], buf_v)

            def p4(i, carry):
                sl = pl.ds(i * 16, 16)
                acc_v[sl] = acc_v[sl] + buf_v[sl]
                return carry
            lax.fori_loop(0, vslc // 16, p4, None)
        pltpu.sync_copy(acc_v, out_hbm.at[pl.ds(c * _V + t * vslc, vslc)])


_select_kernel = pl.kernel(
    _select_body,
    out_type=jax.ShapeDtypeStruct((3 * _V,), jnp.float32),
    mesh=plsc.VectorSubcoreMesh(
        core_axis_name="c", subcore_axis_name="s", num_cores=1),
    compiler_params=pltpu.CompilerParams(needs_layout_passes=False),
    scratch_types=[
        pltpu.VMEM((_SEG * 3,), jnp.float32),
        pltpu.VMEM((_SEG,), jnp.int32),
        pltpu.VMEM((_SEG,), jnp.int32),
        pltpu.VMEM((_V,), jnp.int32),
        pltpu.VMEM((_V,), jnp.int32),
        pltpu.VMEM((_V,), jnp.int32),
        pltpu.VMEM((_V,), jnp.int32),
        pltpu.VMEM((_V,), jnp.int32),
        pltpu.VMEM((16,), jnp.float32),
        pltpu.VMEM((_NSUB * 16,), jnp.float32),
        pltpu.VMEM((_V,), jnp.float32),
        pltpu.VMEM((_V,), jnp.float32),
        pltpu.VMEM((_V,), jnp.float32),
        pltpu.VMEM((_V // _NSUB,), jnp.float32),
        pltpu.VMEM((_V // _NSUB,), jnp.float32),
        pltpu.VMEM_SHARED((_NSUB * _V,), jnp.int32),
        pltpu.VMEM_SHARED((_NSUB * 3 * _V,), jnp.float32),
        pltpu.VMEM_SHARED((_NSUB * 16,), jnp.float32),
    ],
)


def _knn_body(x_ref, vt_ref, p2v_ref, dist_ref):
    x = x_ref[...]
    vt = vt_ref[...]
    psq = jnp.sum(x * x, axis=1, keepdims=True)
    vsq = jnp.sum(vt * vt, axis=0, keepdims=True)
    pv2 = jnp.dot(x, -2.0 * vt, preferred_element_type=jnp.float32)
    d = pv2 + vsq
    vals = [d[:, 128 * k:128 * (k + 1)] for k in range(_V // 128)]
    idxs = [jnp.full((_PBLK, 128), float(k), jnp.float32)
            for k in range(_V // 128)]
    while len(vals) > 1:
        nv, ni = [], []
        for a in range(0, len(vals), 2):
            lt = vals[a + 1] < vals[a]
            nv.append(jnp.where(lt, vals[a + 1], vals[a]))
            ni.append(jnp.where(lt, idxs[a + 1], idxs[a]))
        vals, idxs = nv, ni
    best, bidx = vals[0], idxs[0]
    mval = jnp.min(best, axis=1, keepdims=True)
    lane = lax.broadcasted_iota(
        jnp.int32, (_PBLK, 128), 1).astype(jnp.float32)
    gidx = bidx * 128.0 + lane
    cand = jnp.where(best == mval, gidx, jnp.float32(_V))
    p2v_ref[...] = jnp.min(cand, axis=1).astype(jnp.int32).reshape(1, 1, _PBLK)
    dist_ref[...] = (mval + psq).reshape(1, 1, _PBLK)


_knn_kernel = pl.pallas_call(
    _knn_body,
    grid=(_N // _PBLK,),
    in_specs=[
        pl.BlockSpec((_PBLK, 3), lambda i: (i, 0)),
        pl.BlockSpec((3, _V), lambda i: (0, 0)),
    ],
    out_specs=[
        pl.BlockSpec((1, 1, _PBLK), lambda i: (i, 0, 0)),
        pl.BlockSpec((1, 1, _PBLK), lambda i: (i, 0, 0)),
    ],
    out_shape=[
        jax.ShapeDtypeStruct((_N // _PBLK, 1, _PBLK), jnp.int32),
        jax.ShapeDtypeStruct((_N // _PBLK, 1, _PBLK), jnp.float32),
    ],
)


def kernel(xyz):
    pick = jnp.linspace(0.0, _N - 1, _V).astype(jnp.int32)
    vt = _select_kernel(xyz.reshape(3 * _N), pick).reshape(3, _V)
    p2v, min_dist = _knn_kernel(xyz, vt)
    return vt.T, p2v.reshape(_N), min_dist.reshape(_N)


# speedup vs baseline: 1.1362x; 1.1362x over previous
import functools

import numpy as np
import jax
import jax.numpy as jnp
from jax import lax
from jax.experimental import pallas as pl
from jax.experimental.pallas import tpu as pltpu
from jax.experimental.pallas import tpu_sc as plsc

_N = 131072
_V = 4096
_R = 16
_NSUB = 16
_SEG = _N // _NSUB
_GRP = _SEG // 16
_PBLK = 2048
_PICK_SCALE = np.float32(_V - 1) / np.float32(_N - 1)


def _keys_body(xt_ref, keys_ref):
    x = xt_ref[...]
    mn = jnp.min(x, axis=1, keepdims=True)
    mx = jnp.max(x, axis=1, keepdims=True)
    cell = (mx - mn) / _R + 1e-12
    q = (x - mn) / cell
    vox = jnp.clip(q.astype(jnp.int32), 0, _R - 1)
    keys_ref[...] = (vox[0:1, :] * (_R * _R) + vox[1:2, :] * _R + vox[2:3, :])


_keys_kernel = pl.pallas_call(
    _keys_body,
    out_shape=jax.ShapeDtypeStruct((1, _N), jnp.int32),
)


def _sc_zero(ref, n):
    def body(i, carry):
        ref[pl.ds(i * 16, 16)] = jnp.zeros((16,), ref.dtype)
        return carry
    lax.fori_loop(0, n // 16, body, None)


def _select_body(keys_hbm, xt_hbm, pick_hbm, out_hbm,
                 keys_v, locc_v, hist_v, base_v, tot_v, tmp_v, pick_v,
                 xx_v, xy_v, xz_v, vx_v, vy_v, vz_v, acc_v, buf_v,
                 hist_sh, vert_sh):
    t = lax.axis_index("s")
    seg0 = t * _SEG

    pltpu.sync_copy(keys_hbm.at[pl.ds(seg0, _SEG)], keys_v)
    pltpu.sync_copy(pick_hbm, pick_v)
    pltpu.sync_copy(xt_hbm.at[pl.ds(seg0, _SEG)], xx_v)
    pltpu.sync_copy(xt_hbm.at[pl.ds(_N + seg0, _SEG)], xy_v)
    pltpu.sync_copy(xt_hbm.at[pl.ds(2 * _N + seg0, _SEG)], xz_v)

    _sc_zero(hist_v, _V)
    _sc_zero(base_v, _V)
    _sc_zero(tot_v, _V)
    _sc_zero(vx_v, _V)
    _sc_zero(vy_v, _V)
    _sc_zero(vz_v, _V)

    def p1(g, carry):
        sl = pl.ds(g * 16, 16)
        k16 = keys_v[sl]
        gath = plsc.load_gather(hist_v, [k16])
        cnt, last = plsc.scan_count(k16)
        locc_v[sl] = gath + cnt - 1
        plsc.store_scatter(hist_v, [k16], gath + cnt, mask=last)
        return carry
    lax.fori_loop(0, _GRP, p1, None)

    pltpu.sync_copy(hist_v, hist_sh.at[pl.ds(t * _V, _V)])
    plsc.subcore_barrier()
    for t2 in range(_NSUB):
        pltpu.sync_copy(hist_sh.at[pl.ds(t2 * _V, _V)], tmp_v)

        def p2(i, carry, _t2=t2):
            sl = pl.ds(i * 16, 16)
            h = tmp_v[sl]
            tot_v[sl] = tot_v[sl] + h
            base_v[sl] = base_v[sl] + jnp.where(_t2 < t, h, 0)
            return carry
        lax.fori_loop(0, _V // 16, p2, None)

    def p2c(i, carry):
        sl = pl.ds(i * 16, 16)
        v = tot_v[sl]
        cs = plsc.cumsum(v)
        base_v[sl] = base_v[sl] + cs - v + carry
        return carry + jnp.sum(v)
    lax.fori_loop(0, _V // 16, p2c, jnp.int32(0))

    def p3(g, carry):
        sl = pl.ds(g * 16, 16)
        k16 = keys_v[sl]
        rank = plsc.load_gather(base_v, [k16]) + locc_v[sl]
        i0 = (rank.astype(jnp.float32) * _PICK_SCALE).astype(jnp.int32)
        hit_any = jnp.zeros((16,), jnp.bool_)
        isel = jnp.zeros((16,), jnp.int32)
        for dd in (-1, 0, 1):
            ic = jnp.clip(i0 + dd, 0, _V - 1)
            pk = plsc.load_gather(pick_v, [ic])
            hit = jnp.logical_and(pk == rank, jnp.logical_not(hit_any))
            isel = jnp.where(hit, ic, isel)
            hit_any = jnp.logical_or(hit_any, hit)
        plsc.store_scatter(vx_v, [isel], xx_v[sl], mask=hit_any)
        plsc.store_scatter(vy_v, [isel], xy_v[sl], mask=hit_any)
        plsc.store_scatter(vz_v, [isel], xz_v[sl], mask=hit_any)
        return carry
    lax.fori_loop(0, _GRP, p3, None)

    pltpu.sync_copy(vx_v, vert_sh.at[pl.ds((t * 3 + 0) * _V, _V)])
    pltpu.sync_copy(vy_v, vert_sh.at[pl.ds((t * 3 + 1) * _V, _V)])
    pltpu.sync_copy(vz_v, vert_sh.at[pl.ds((t * 3 + 2) * _V, _V)])
    plsc.subcore_barrier()
    vslc = _V // _NSUB
    for c in range(3):
        _sc_zero(acc_v, vslc)
        for t2 in range(_NSUB):
            pltpu.sync_copy(
                vert_sh.at[pl.ds((t2 * 3 + c) * _V + t * vslc, vslc)], buf_v)

            def p4(i, carry):
                sl = pl.ds(i * 16, 16)
                acc_v[sl] = acc_v[sl] + buf_v[sl]
                return carry
            lax.fori_loop(0, vslc // 16, p4, None)
        pltpu.sync_copy(acc_v, out_hbm.at[pl.ds(c * _V + t * vslc, vslc)])


_select_kernel = pl.kernel(
    _select_body,
    out_type=jax.ShapeDtypeStruct((3 * _V,), jnp.float32),
    mesh=plsc.VectorSubcoreMesh(
        core_axis_name="c", subcore_axis_name="s", num_cores=1),
    compiler_params=pltpu.CompilerParams(needs_layout_passes=False),
    scratch_types=[
        pltpu.VMEM((_SEG,), jnp.int32),
        pltpu.VMEM((_SEG,), jnp.int32),
        pltpu.VMEM((_V,), jnp.int32),
        pltpu.VMEM((_V,), jnp.int32),
        pltpu.VMEM((_V,), jnp.int32),
        pltpu.VMEM((_V,), jnp.int32),
        pltpu.VMEM((_V,), jnp.int32),
        pltpu.VMEM((_SEG,), jnp.float32),
        pltpu.VMEM((_SEG,), jnp.float32),
        pltpu.VMEM((_SEG,), jnp.float32),
        pltpu.VMEM((_V,), jnp.float32),
        pltpu.VMEM((_V,), jnp.float32),
        pltpu.VMEM((_V,), jnp.float32),
        pltpu.VMEM((_V // _NSUB,), jnp.float32),
        pltpu.VMEM((_V // _NSUB,), jnp.float32),
        pltpu.VMEM_SHARED((_NSUB * _V,), jnp.int32),
        pltpu.VMEM_SHARED((_NSUB * 3 * _V,), jnp.float32),
    ],
)


def _knn_body(x_ref, vt_ref, p2v_ref, dist_ref):
    x = x_ref[...]
    vt = vt_ref[...]
    psq = jnp.sum(x * x, axis=1, keepdims=True)
    vsq = jnp.sum(vt * vt, axis=0, keepdims=True)
    pv2 = jnp.dot(x, -2.0 * vt, preferred_element_type=jnp.float32)
    d = pv2 + vsq
    vals = [d[:, 128 * k:128 * (k + 1)] for k in range(_V // 128)]
    idxs = [jnp.full((_PBLK, 128), float(k), jnp.float32)
            for k in range(_V // 128)]
    while len(vals) > 1:
        nv, ni = [], []
        for a in range(0, len(vals), 2):
            lt = vals[a + 1] < vals[a]
            nv.append(jnp.where(lt, vals[a + 1], vals[a]))
            ni.append(jnp.where(lt, idxs[a + 1], idxs[a]))
        vals, idxs = nv, ni
    best, bidx = vals[0], idxs[0]
    mval = jnp.min(best, axis=1, keepdims=True)
    lane = lax.broadcasted_iota(
        jnp.int32, (_PBLK, 128), 1).astype(jnp.float32)
    gidx = bidx * 128.0 + lane
    cand = jnp.where(best == mval, gidx, jnp.float32(_V))
    p2v_ref[...] = jnp.min(cand, axis=1).astype(jnp.int32).reshape(1, 1, _PBLK)
    dist_ref[...] = (mval + psq).reshape(1, 1, _PBLK)


_knn_kernel = pl.pallas_call(
    _knn_body,
    grid=(_N // _PBLK,),
    in_specs=[
        pl.BlockSpec((_PBLK, 3), lambda i: (i, 0)),
        pl.BlockSpec((3, _V), lambda i: (0, 0)),
    ],
    out_specs=[
        pl.BlockSpec((1, 1, _PBLK), lambda i: (i, 0, 0)),
        pl.BlockSpec((1, 1, _PBLK), lambda i: (i, 0, 0)),
    ],
    out_shape=[
        jax.ShapeDtypeStruct((_N // _PBLK, 1, _PBLK), jnp.int32),
        jax.ShapeDtypeStruct((_N // _PBLK, 1, _PBLK), jnp.float32),
    ],
)


def kernel(xyz):
    xt = xyz.T
    keys = _keys_kernel(xt).reshape(_N)
    pick = jnp.linspace(0.0, _N - 1, _V).astype(jnp.int32)
    vt = _select_kernel(keys, xt.reshape(3 * _N), pick).reshape(3, _V)
    p2v, min_dist = _knn_kernel(xyz, vt)
    return vt.T, p2v.reshape(_N), min_dist.reshape(_N)


# speedup vs baseline: 1.1682x; 1.0282x over previous
import functools

import numpy as np
import jax
import jax.numpy as jnp
from jax import lax
from jax.experimental import pallas as pl
from jax.experimental.pallas import tpu as pltpu
from jax.experimental.pallas import tpu_sc as plsc

_N = 131072
_V = 4096
_R = 16
_NSUB = 16
_SEG = _N // _NSUB
_GRP = _SEG // 16
_PBLK = 2048
_PICK_SCALE = np.float32(_V - 1) / np.float32(_N - 1)


def _keys_body(xt_ref, keys_ref):
    x = xt_ref[...]
    mn = jnp.min(x, axis=1, keepdims=True)
    mx = jnp.max(x, axis=1, keepdims=True)
    cell = (mx - mn) / _R + 1e-12
    q = (x - mn) / cell
    vox = jnp.clip(q.astype(jnp.int32), 0, _R - 1)
    keys_ref[...] = (vox[0:1, :] * (_R * _R) + vox[1:2, :] * _R + vox[2:3, :])


_keys_kernel = pl.pallas_call(
    _keys_body,
    out_shape=jax.ShapeDtypeStruct((1, _N), jnp.int32),
)


def _sc_zero(ref, n):
    def body(i, carry):
        ref[pl.ds(i * 16, 16)] = jnp.zeros((16,), ref.dtype)
        return carry
    lax.fori_loop(0, n // 16, body, None)


def _select_body(keys_hbm, xt_hbm, pick_hbm, out_hbm,
                 keys_v, locc_v, hist_v, base_v, tmp_v, pick_v,
                 h256_v, run_v, pfx_v,
                 xx_v, xy_v, xz_v, vx_v, vy_v, vz_v, acc_v, buf_v,
                 hist_sh, vert_sh, base_sh, stot_sh):
    t = lax.axis_index("s")
    seg0 = t * _SEG
    vslc = _V // _NSUB

    pltpu.sync_copy(keys_hbm.at[pl.ds(seg0, _SEG)], keys_v)
    pltpu.sync_copy(pick_hbm, pick_v)
    pltpu.sync_copy(xt_hbm.at[pl.ds(seg0, _SEG)], xx_v)
    pltpu.sync_copy(xt_hbm.at[pl.ds(_N + seg0, _SEG)], xy_v)
    pltpu.sync_copy(xt_hbm.at[pl.ds(2 * _N + seg0, _SEG)], xz_v)

    _sc_zero(hist_v, _V)
    _sc_zero(vx_v, _V)
    _sc_zero(vy_v, _V)
    _sc_zero(vz_v, _V)
    _sc_zero(run_v, vslc)

    def p1(g, carry):
        sl = pl.ds(g * 16, 16)
        k16 = keys_v[sl]
        gath = plsc.load_gather(hist_v, [k16])
        cnt, last = plsc.scan_count(k16)
        locc_v[sl] = gath + cnt - 1
        plsc.store_scatter(hist_v, [k16], gath + cnt, mask=last)
        return carry
    lax.fori_loop(0, _GRP, p1, None)

    pltpu.sync_copy(hist_v, hist_sh.at[pl.ds(t * _V, _V)])
    plsc.subcore_barrier()
    for t2 in range(_NSUB):
        pltpu.sync_copy(hist_sh.at[pl.ds(t2 * _V + t * vslc, vslc)], h256_v)

        def p2(i, carry, _t2=t2):
            sl = pl.ds(i * 16, 16)
            tmp_v[pl.ds(_t2 * vslc + i * 16, 16)] = run_v[sl]
            run_v[sl] = run_v[sl] + h256_v[sl]
            return carry
        lax.fori_loop(0, vslc // 16, p2, None)

    def p2c(i, carry):
        sl = pl.ds(i * 16, 16)
        v = run_v[sl]
        cs = plsc.cumsum(v)
        pfx_v[sl] = cs - v + carry
        return carry + jnp.sum(v)
    stot = lax.fori_loop(0, vslc // 16, p2c, jnp.int32(0))
    h256_v[pl.ds(0, 16)] = jnp.full((16,), 1, jnp.int32) * stot
    pltpu.sync_copy(h256_v.at[pl.ds(0, 16)], stot_sh.at[pl.ds(t * 16, 16)])
    plsc.subcore_barrier()
    goff = jnp.zeros((16,), jnp.int32)
    for t2 in range(_NSUB):
        pltpu.sync_copy(stot_sh.at[pl.ds(t2 * 16, 16)], h256_v.at[pl.ds(0, 16)])
        goff = goff + jnp.where(t2 < t, h256_v[pl.ds(0, 16)], 0)
    for t2 in range(_NSUB):
        def p2w(i, carry, _t2=t2):
            sl = pl.ds(i * 16, 16)
            h256_v[sl] = tmp_v[pl.ds(_t2 * vslc + i * 16, 16)] + pfx_v[sl] + goff
            return carry
        lax.fori_loop(0, vslc // 16, p2w, None)
        pltpu.sync_copy(h256_v, base_sh.at[pl.ds(t2 * _V + t * vslc, vslc)])
    plsc.subcore_barrier()
    pltpu.sync_copy(base_sh.at[pl.ds(t * _V, _V)], base_v)

    def p3(g, carry):
        sl = pl.ds(g * 16, 16)
        k16 = keys_v[sl]
        rank = plsc.load_gather(base_v, [k16]) + locc_v[sl]
        i0 = (rank.astype(jnp.float32) * _PICK_SCALE).astype(jnp.int32)
        hit_any = jnp.zeros((16,), jnp.bool_)
        isel = jnp.zeros((16,), jnp.int32)
        for dd in (-1, 0, 1):
            ic = jnp.clip(i0 + dd, 0, _V - 1)
            pk = plsc.load_gather(pick_v, [ic])
            hit = jnp.logical_and(pk == rank, jnp.logical_not(hit_any))
            isel = jnp.where(hit, ic, isel)
            hit_any = jnp.logical_or(hit_any, hit)
        plsc.store_scatter(vx_v, [isel], xx_v[sl], mask=hit_any)
        plsc.store_scatter(vy_v, [isel], xy_v[sl], mask=hit_any)
        plsc.store_scatter(vz_v, [isel], xz_v[sl], mask=hit_any)
        return carry
    lax.fori_loop(0, _GRP, p3, None)

    pltpu.sync_copy(vx_v, vert_sh.at[pl.ds((t * 3 + 0) * _V, _V)])
    pltpu.sync_copy(vy_v, vert_sh.at[pl.ds((t * 3 + 1) * _V, _V)])
    pltpu.sync_copy(vz_v, vert_sh.at[pl.ds((t * 3 + 2) * _V, _V)])
    plsc.subcore_barrier()
    vslc = _V // _NSUB
    for c in range(3):
        _sc_zero(acc_v, vslc)
        for t2 in range(_NSUB):
            pltpu.sync_copy(
                vert_sh.at[pl.ds((t2 * 3 + c) * _V + t * vslc, vslc)], buf_v)

            def p4(i, carry):
                sl = pl.ds(i * 16, 16)
                acc_v[sl] = acc_v[sl] + buf_v[sl]
                return carry
            lax.fori_loop(0, vslc // 16, p4, None)
        pltpu.sync_copy(acc_v, out_hbm.at[pl.ds(c * _V + t * vslc, vslc)])


_select_kernel = pl.kernel(
    _select_body,
    out_type=jax.ShapeDtypeStruct((3 * _V,), jnp.float32),
    mesh=plsc.VectorSubcoreMesh(
        core_axis_name="c", subcore_axis_name="s", num_cores=1),
    compiler_params=pltpu.CompilerParams(needs_layout_passes=False),
    scratch_types=[
        pltpu.VMEM((_SEG,), jnp.int32),
        pltpu.VMEM((_SEG,), jnp.int32),
        pltpu.VMEM((_V,), jnp.int32),
        pltpu.VMEM((_V,), jnp.int32),
        pltpu.VMEM((_V,), jnp.int32),
        pltpu.VMEM((_V,), jnp.int32),
        pltpu.VMEM((_V // _NSUB,), jnp.int32),
        pltpu.VMEM((_V // _NSUB,), jnp.int32),
        pltpu.VMEM((_V // _NSUB,), jnp.int32),
        pltpu.VMEM((_SEG,), jnp.float32),
        pltpu.VMEM((_SEG,), jnp.float32),
        pltpu.VMEM((_SEG,), jnp.float32),
        pltpu.VMEM((_V,), jnp.float32),
        pltpu.VMEM((_V,), jnp.float32),
        pltpu.VMEM((_V,), jnp.float32),
        pltpu.VMEM((_V // _NSUB,), jnp.float32),
        pltpu.VMEM((_V // _NSUB,), jnp.float32),
        pltpu.VMEM_SHARED((_NSUB * _V,), jnp.int32),
        pltpu.VMEM_SHARED((_NSUB * 3 * _V,), jnp.float32),
        pltpu.VMEM_SHARED((_NSUB * _V,), jnp.int32),
        pltpu.VMEM_SHARED((_NSUB * 16,), jnp.int32),
    ],
)


def _knn_body(x_ref, vt_ref, p2v_ref, dist_ref):
    x = x_ref[...]
    vt = vt_ref[...]
    psq = jnp.sum(x * x, axis=1, keepdims=True)
    vsq = jnp.sum(vt * vt, axis=0, keepdims=True)
    pv2 = jnp.dot(x, -2.0 * vt, preferred_element_type=jnp.float32)
    d = pv2 + vsq
    vals = [d[:, 128 * k:128 * (k + 1)] for k in range(_V // 128)]
    idxs = [jnp.full((_PBLK, 128), float(k), jnp.float32)
            for k in range(_V // 128)]
    while len(vals) > 1:
        nv, ni = [], []
        for a in range(0, len(vals), 2):
            lt = vals[a + 1] < vals[a]
            nv.append(jnp.where(lt, vals[a + 1], vals[a]))
            ni.append(jnp.where(lt, idxs[a + 1], idxs[a]))
        vals, idxs = nv, ni
    best, bidx = vals[0], idxs[0]
    mval = jnp.min(best, axis=1, keepdims=True)
    lane = lax.broadcasted_iota(
        jnp.int32, (_PBLK, 128), 1).astype(jnp.float32)
    gidx = bidx * 128.0 + lane
    cand = jnp.where(best == mval, gidx, jnp.float32(_V))
    p2v_ref[...] = jnp.min(cand, axis=1).astype(jnp.int32).reshape(1, 1, _PBLK)
    dist_ref[...] = (mval + psq).reshape(1, 1, _PBLK)


_knn_kernel = pl.pallas_call(
    _knn_body,
    grid=(_N // _PBLK,),
    in_specs=[
        pl.BlockSpec((_PBLK, 3), lambda i: (i, 0)),
        pl.BlockSpec((3, _V), lambda i: (0, 0)),
    ],
    out_specs=[
        pl.BlockSpec((1, 1, _PBLK), lambda i: (i, 0, 0)),
        pl.BlockSpec((1, 1, _PBLK), lambda i: (i, 0, 0)),
    ],
    out_shape=[
        jax.ShapeDtypeStruct((_N // _PBLK, 1, _PBLK), jnp.int32),
        jax.ShapeDtypeStruct((_N // _PBLK, 1, _PBLK), jnp.float32),
    ],
)


def kernel(xyz):
    xt = xyz.T
    keys = _keys_kernel(xt).reshape(_N)
    pick = jnp.linspace(0.0, _N - 1, _V).astype(jnp.int32)
    vt = _select_kernel(keys, xt.reshape(3 * _N), pick).reshape(3, _V)
    p2v, min_dist = _knn_kernel(xyz, vt)
    return vt.T, p2v.reshape(_N), min_dist.reshape(_N)
